# baseline (speedup 1.0000x reference)
import functools

import jax
import jax.numpy as jnp
from jax import lax
from jax.experimental import pallas as pl
from jax.experimental.pallas import tpu as pltpu
from jax.experimental.pallas import tpu_sc as plsc

_B, _C, _N = 4, 64, 32768
_M, _K = 8192, 16
_NC, _NS, _L = 2, 16, 16
_NW = _NC * _NS
_ROWS = _B * _M
_RPW = _ROWS // _NW
_R = 32
_STEPS = _RPW // _R
_IDXW = 128
_G = (_R * _K) // _IDXW
_IDXR = _RPW * _K // _IDXW


def _gather_max(table, idx):
  mesh = plsc.VectorSubcoreMesh(
      core_axis_name="c", subcore_axis_name="s",
      num_cores=_NC, num_subcores=_NS)

  @functools.partial(
      pl.kernel,
      out_type=jax.ShapeDtypeStruct((_B, _C, _M), jnp.float32),
      mesh=mesh,
      scratch_types=[
          pltpu.VMEM((_IDXR, _IDXW), jnp.int32),
          pltpu.VMEM((_R * _K, _C), jnp.float32),
          pltpu.VMEM((_R * _K, _C), jnp.float32),
          pltpu.VMEM((_C, _R), jnp.float32),
          pltpu.VMEM((_C, _R), jnp.float32),
          pltpu.SemaphoreType.DMA,
          pltpu.SemaphoreType.DMA,
          pltpu.SemaphoreType.DMA,
          pltpu.SemaphoreType.DMA,
      ],
      compiler_params=pltpu.CompilerParams(
          use_tc_tiling_on_sc=False, needs_layout_passes=False),
  )
  def body(table_hbm, idx_hbm, out_hbm, idx_v, rows_v0, rows_v1,
           outT_v0, outT_v1, sg0, sg1, so0, so1):
    wid = lax.axis_index("s") * _NC + lax.axis_index("c")
    wbase = wid * _RPW
    batch = wbase // _M
    mw = wbase % _M
    sg = (sg0, sg1)
    so = (so0, so1)
    rows_v = (rows_v0, rows_v1)
    outT_v = (outT_v0, outT_v1)
    iotaK = lax.iota(jnp.int32, _L) * _K

    pltpu.sync_copy(idx_hbm.at[wid], idx_v)

    boffv = jnp.full((_L,), 0, jnp.int32) + (wbase // _M) * _N

    @pl.loop(0, _IDXR)
    def _adjust(r):
      for j in range(_IDXW // _L):
        sl = pl.ds(j * _L, _L)
        idx_v[r, sl] = idx_v[r, sl] + boffv

    def fire(s, P):
      for g in range(_G):
        pltpu.async_copy(
            table_hbm.at[idx_v.at[s * _G + g]],
            rows_v[P].at[pl.ds(g * _IDXW, _IDXW)],
            sg[P])

    def drain(s, P):
      for g in range(_G):
        pltpu.make_async_copy(
            table_hbm.at[idx_v.at[s * _G + g]],
            rows_v[P].at[pl.ds(g * _IDXW, _IDXW)],
            sg[P]).wait()

    def compute(P):
      rv = rows_v[P]
      ot = outT_v[P]

      @pl.loop(0, _C)
      def _col(c):
        csplat = jnp.full((_L,), 0, jnp.int32) + c
        for mb in range(_R // _L):
          rbase = iotaK + mb * _L * _K
          acc = plsc.load_gather(rv, [rbase, csplat])
          for kk in range(1, _K):
            acc = jnp.maximum(
                acc, plsc.load_gather(rv, [rbase + kk, csplat]))
          ot[c, pl.ds(mb * _L, _L)] = acc

    def store(s, P):
      m0 = pl.multiple_of(mw + s * _R, _R)
      pltpu.async_copy(
          outT_v[P], out_hbm.at[batch, :, pl.ds(m0, _R)], so[P])

    def drain_store(s, P):
      m0 = pl.multiple_of(mw + s * _R, _R)
      pltpu.make_async_copy(
          outT_v[P], out_hbm.at[batch, :, pl.ds(m0, _R)], so[P]).wait()

    fire(0, 0)

    @pl.loop(0, _STEPS // 2)
    def _pair(p):
      s0 = p * 2
      fire(s0 + 1, 1)
      drain(s0, 0)

      @pl.when(p > 0)
      def _():
        drain_store(s0 - 2, 0)

      compute(0)
      store(s0, 0)

      @pl.when(p < _STEPS // 2 - 1)
      def _():
        fire(s0 + 2, 0)

      drain(s0 + 1, 1)

      @pl.when(p > 0)
      def _():
        drain_store(s0 - 1, 1)

      compute(1)
      store(s0 + 1, 1)

    drain_store(_STEPS - 2, 0)
    drain_store(_STEPS - 1, 1)

  return body(table, idx)


def kernel(input, points, support_points, indices):
  table = input.transpose(0, 2, 1).reshape(_B * _N, _C)
  idx = indices.astype(jnp.int32).reshape(_NW, _IDXR, _IDXW)
  features = _gather_max(table, idx)
  return (features, support_points, indices)


# speedup vs baseline: 3.7150x; 3.7150x over previous
import functools

import jax
import jax.numpy as jnp
from jax import lax
from jax.experimental import pallas as pl
from jax.experimental.pallas import tpu as pltpu
from jax.experimental.pallas import tpu_sc as plsc

_B, _C, _N = 4, 64, 32768
_M, _K = 8192, 16
_NC, _NS, _L = 2, 16, 16
_NW = _NC * _NS
_ROWS = _B * _M
_RPW = _ROWS // _NW
_R = 32
_STEPS = _RPW // _R
_IDXW = 512
_G = (_R * _K) // _IDXW
_IDXR = _RPW * _K // _IDXW


def _gather_max(table, idx):
  mesh = plsc.VectorSubcoreMesh(
      core_axis_name="c", subcore_axis_name="s",
      num_cores=_NC, num_subcores=_NS)

  @functools.partial(
      pl.kernel,
      out_type=jax.ShapeDtypeStruct((_ROWS, _C), jnp.float32),
      mesh=mesh,
      scratch_types=[
          pltpu.VMEM((_IDXR, _IDXW), jnp.int32),
          pltpu.VMEM((2, _R * _K, _C), jnp.float32),
          pltpu.VMEM((2, _R, _C), jnp.float32),
          pltpu.SemaphoreType.DMA,
          pltpu.SemaphoreType.DMA,
          pltpu.SemaphoreType.DMA,
          pltpu.SemaphoreType.DMA,
      ],
      compiler_params=pltpu.CompilerParams(use_tc_tiling_on_sc=False),
  )
  def body(table_hbm, idx_hbm, out_hbm, idx_v, rows_v, out_v, sg0, sg1, so0, so1):
    wid = lax.axis_index("s") * _NC + lax.axis_index("c")
    wbase = wid * _RPW
    sg = (sg0, sg1)
    so = (so0, so1)

    pltpu.sync_copy(idx_hbm.at[wid], idx_v)

    boffv = jnp.full((_L,), 0, jnp.int32) + (wbase // _M) * _N

    @pl.loop(0, _IDXR)
    def _adjust(r):
      for j in range(_IDXW // _L):
        sl = pl.ds(j * _L, _L)
        idx_v[r, sl] = idx_v[r, sl] + boffv

    def fire(s, P):
      for g in range(_G):
        pltpu.async_copy(
            table_hbm.at[idx_v.at[s * _G + g]],
            rows_v.at[P, pl.ds(g * _IDXW, _IDXW)],
            sg[P])

    def drain(s, P):
      for g in range(_G):
        pltpu.make_async_copy(
            table_hbm.at[idx_v.at[s * _G + g]],
            rows_v.at[P, pl.ds(g * _IDXW, _IDXW)],
            sg[P]).wait()

    def compute(P):
      @pl.loop(0, _R // 2)
      def _row(r2):
        for u in range(2):
          r = r2 * 2 + u
          rb = r * _K
          for j in range(_C // _L):
            sl = pl.ds(j * _L, _L)
            acc = rows_v[P, rb, sl]
            for kk in range(1, _K):
              acc = jnp.maximum(acc, rows_v[P, rb + kk, sl])
            out_v[P, r, sl] = acc

    def store(s, P):
      base = pl.multiple_of(wbase + s * _R, _R)
      pltpu.async_copy(out_v.at[P], out_hbm.at[pl.ds(base, _R)], so[P])

    def drain_store(s, P):
      base = pl.multiple_of(wbase + s * _R, _R)
      pltpu.make_async_copy(
          out_v.at[P], out_hbm.at[pl.ds(base, _R)], so[P]).wait()

    fire(0, 0)

    @pl.loop(0, _STEPS // 2)
    def _pair(p):
      s0 = p * 2
      fire(s0 + 1, 1)
      drain(s0, 0)

      @pl.when(p > 0)
      def _():
        drain_store(s0 - 2, 0)

      compute(0)
      store(s0, 0)

      @pl.when(p < _STEPS // 2 - 1)
      def _():
        fire(s0 + 2, 0)

      drain(s0 + 1, 1)

      @pl.when(p > 0)
      def _():
        drain_store(s0 - 1, 1)

      compute(1)
      store(s0 + 1, 1)

    drain_store(_STEPS - 2, 0)
    drain_store(_STEPS - 1, 1)

  return body(table, idx)


def kernel(input, points, support_points, indices):
  table = input.transpose(0, 2, 1).reshape(_B * _N, _C)
  idx = indices.astype(jnp.int32).reshape(_NW, _IDXR, _IDXW)
  out = _gather_max(table, idx)
  features = out.reshape(_B, _M, _C).transpose(0, 2, 1)
  return (features, support_points, indices)


# speedup vs baseline: 4.0928x; 1.1017x over previous
import functools

import jax
import jax.numpy as jnp
from jax import lax
from jax.experimental import pallas as pl
from jax.experimental.pallas import tpu as pltpu
from jax.experimental.pallas import tpu_sc as plsc

_B, _C, _N = 4, 64, 32768
_M, _K = 8192, 16
_NC, _NS, _L = 2, 16, 16
_NW = _NC * _NS
_ROWS = _B * _M
_RPW = _ROWS // _NW
_R = 32
_STEPS = _RPW // _R
_IDXW = 512
_G = (_R * _K) // _IDXW
_IDXR = _RPW * _K // _IDXW


def _gather_max(table, idx):
  mesh = plsc.VectorSubcoreMesh(
      core_axis_name="c", subcore_axis_name="s",
      num_cores=_NC, num_subcores=_NS)

  @functools.partial(
      pl.kernel,
      out_type=jax.ShapeDtypeStruct((_ROWS, _C), jnp.float32),
      mesh=mesh,
      scratch_types=[
          pltpu.VMEM((2, _K, _R), jnp.int32),
          pltpu.VMEM((2, _K * _R), jnp.int32),
          pltpu.VMEM((2, _R * _K, _C), jnp.float32),
          pltpu.VMEM((2, _R, _C), jnp.float32),
          pltpu.SemaphoreType.DMA,
          pltpu.SemaphoreType.DMA,
          pltpu.SemaphoreType.DMA,
          pltpu.SemaphoreType.DMA,
          pltpu.SemaphoreType.DMA,
          pltpu.SemaphoreType.DMA,
      ],
      compiler_params=pltpu.CompilerParams(use_tc_tiling_on_sc=False),
  )
  def body(table_hbm, idx_hbm, out_hbm, idx_v, idx_l, rows_v, out_v,
           sg0, sg1, so0, so1, si0, si1):
    wid = lax.axis_index("s") * _NC + lax.axis_index("c")
    wbase = wid * _RPW
    batch = wbase // _M
    mw = wbase % _M
    sg = (sg0, sg1)
    so = (so0, so1)
    si = (si0, si1)
    boffv = jnp.full((_L,), 0, jnp.int32) + batch * _N

    def stage_idx(s, P):
      m0 = pl.multiple_of(mw + s * _R, _R)
      pltpu.async_copy(
          idx_hbm.at[batch, :, pl.ds(m0, _R)], idx_v.at[P], si[P])

    def fire(s, P):
      m0 = pl.multiple_of(mw + s * _R, _R)
      pltpu.make_async_copy(
          idx_hbm.at[batch, :, pl.ds(m0, _R)], idx_v.at[P], si[P]).wait()
      for k in range(_K):
        for h in range(_R // _L):
          sl = pl.ds(h * _L, _L)
          idx_l[P, pl.ds(k * _R + h * _L, _L)] = idx_v[P, k, sl] + boffv
      pltpu.async_copy(
          table_hbm.at[idx_l.at[P]], rows_v.at[P], sg[P])

    def drain(P):
      pltpu.make_async_copy(
          table_hbm.at[idx_l.at[P]], rows_v.at[P], sg[P]).wait()

    def compute(P):
      @pl.loop(0, _R // 2)
      def _row(r2):
        for u in range(2):
          r = r2 * 2 + u
          for j in range(_C // _L):
            sl = pl.ds(j * _L, _L)
            acc = rows_v[P, r, sl]
            for kk in range(1, _K):
              acc = jnp.maximum(acc, rows_v[P, kk * _R + r, sl])
            out_v[P, r, sl] = acc

    def store(s, P):
      base = pl.multiple_of(wbase + s * _R, _R)
      pltpu.async_copy(out_v.at[P], out_hbm.at[pl.ds(base, _R)], so[P])

    def drain_store(s, P):
      base = pl.multiple_of(wbase + s * _R, _R)
      pltpu.make_async_copy(
          out_v.at[P], out_hbm.at[pl.ds(base, _R)], so[P]).wait()

    stage_idx(0, 0)
    stage_idx(1, 1)
    fire(0, 0)

    @pl.loop(0, _STEPS // 2)
    def _pair(p):
      s0 = p * 2
      fire(s0 + 1, 1)

      @pl.when(p < _STEPS // 2 - 1)
      def _():
        stage_idx(s0 + 2, 0)

      drain(0)

      @pl.when(p > 0)
      def _():
        drain_store(s0 - 2, 0)

      compute(0)
      store(s0, 0)

      @pl.when(p < _STEPS // 2 - 1)
      def _():
        fire(s0 + 2, 0)
        stage_idx(s0 + 3, 1)

      drain(1)

      @pl.when(p > 0)
      def _():
        drain_store(s0 - 1, 1)

      compute(1)
      store(s0 + 1, 1)

    drain_store(_STEPS - 2, 0)
    drain_store(_STEPS - 1, 1)

  return body(table, idx)


def kernel(input, points, support_points, indices):
  table = input.transpose(0, 2, 1).reshape(_B * _N, _C)
  idx = indices.transpose(0, 2, 1).astype(jnp.int32)
  out = _gather_max(table, idx)
  features = out.reshape(_B, _M, _C).transpose(0, 2, 1)
  return (features, support_points, indices)
